# baseline (device time: 101578 ns/iter reference)
import jax
import jax.numpy as jnp
from jax import lax
from jax.experimental import pallas as pl
from jax.experimental.pallas import tpu as pltpu

C = 4
ROWS = 2048 // C


def kernel(x, pi):
    def body(pi_ref, x_ref, out_ref, send_buf, recv_buf,
             stage_sem, out_sems, send_sems, recv_sems):
        my_x = lax.axis_index("x")
        my_y = lax.axis_index("y")
        dest_y = pi_ref[my_y]

        @pl.when(dest_y != my_y)
        def _():
            barrier = pltpu.get_barrier_semaphore()
            pl.semaphore_signal(
                barrier, inc=1,
                device_id=(my_x, dest_y),
                device_id_type=pl.DeviceIdType.MESH,
            )

            rdmas = []
            for c in range(C):
                stage = pltpu.make_async_copy(
                    x_ref.at[0, pl.ds(c * ROWS, ROWS), :],
                    send_buf.at[c],
                    stage_sem,
                )
                stage.start()
                stage.wait()
                if c == 0:
                    pl.semaphore_wait(barrier, 1)
                rdma = pltpu.make_async_remote_copy(
                    src_ref=send_buf.at[c],
                    dst_ref=recv_buf.at[c],
                    send_sem=send_sems.at[c],
                    recv_sem=recv_sems.at[c],
                    device_id=(my_x, dest_y),
                    device_id_type=pl.DeviceIdType.MESH,
                )
                rdma.start()
                rdmas.append(rdma)

            out_copies = []
            for c in range(C):
                rdmas[c].wait_recv()
                out_copy = pltpu.make_async_copy(
                    recv_buf.at[c],
                    out_ref.at[0, pl.ds(c * ROWS, ROWS), :],
                    out_sems.at[c],
                )
                out_copy.start()
                out_copies.append(out_copy)

            for c in range(C):
                out_copies[c].wait()
                rdmas[c].wait_send()

        @pl.when(dest_y == my_y)
        def _():
            copy = pltpu.make_async_copy(x_ref, out_ref, stage_sem)
            copy.start()
            copy.wait()

    return pl.pallas_call(
        body,
        out_shape=jax.ShapeDtypeStruct(x.shape, x.dtype),
        in_specs=[
            pl.BlockSpec(memory_space=pltpu.SMEM),
            pl.BlockSpec(memory_space=pl.ANY),
        ],
        out_specs=pl.BlockSpec(memory_space=pl.ANY),
        scratch_shapes=[
            pltpu.VMEM((C, ROWS, 1024), jnp.float32),
            pltpu.VMEM((C, ROWS, 1024), jnp.float32),
            pltpu.SemaphoreType.DMA,
            pltpu.SemaphoreType.DMA((C,)),
            pltpu.SemaphoreType.DMA((C,)),
            pltpu.SemaphoreType.DMA((C,)),
        ],
        compiler_params=pltpu.CompilerParams(collective_id=0),
    )(pi, x)


# device time: 101111 ns/iter; 1.0046x vs baseline; 1.0046x over previous
import jax
import jax.numpy as jnp
from jax import lax
from jax.experimental import pallas as pl
from jax.experimental.pallas import tpu as pltpu


def kernel(x, pi):
    def body(pi_ref, x_ref, out_ref, send_sem, recv_sem):
        my_x = lax.axis_index("x")
        my_y = lax.axis_index("y")
        dest_y = pi_ref[my_y]

        @pl.when(dest_y != my_y)
        def _():
            barrier = pltpu.get_barrier_semaphore()
            pl.semaphore_signal(
                barrier, inc=1,
                device_id=(my_x, dest_y),
                device_id_type=pl.DeviceIdType.MESH,
            )
            pl.semaphore_wait(barrier, 1)
            rdma = pltpu.make_async_remote_copy(
                src_ref=x_ref,
                dst_ref=out_ref,
                send_sem=send_sem,
                recv_sem=recv_sem,
                device_id=(my_x, dest_y),
                device_id_type=pl.DeviceIdType.MESH,
            )
            rdma.start()
            rdma.wait()

        @pl.when(dest_y == my_y)
        def _():
            copy = pltpu.make_async_copy(x_ref, out_ref, send_sem)
            copy.start()
            copy.wait()

    return pl.pallas_call(
        body,
        out_shape=jax.ShapeDtypeStruct(x.shape, x.dtype),
        in_specs=[
            pl.BlockSpec(memory_space=pltpu.SMEM),
            pl.BlockSpec(memory_space=pl.ANY),
        ],
        out_specs=pl.BlockSpec(memory_space=pl.ANY),
        scratch_shapes=[
            pltpu.SemaphoreType.DMA,
            pltpu.SemaphoreType.DMA,
        ],
        compiler_params=pltpu.CompilerParams(collective_id=0),
    )(pi, x)
